# baseline (speedup 1.0000x reference)
import functools

import jax
import jax.numpy as jnp
from jax import lax
from jax.experimental import pallas as pl
from jax.experimental.pallas import tpu as pltpu
from jax.experimental.pallas import tpu_sc as plsc

_NC = 2
_NS = 16
_R = 125
_RP = 128
_NBUF = 4


@functools.lru_cache(maxsize=None)
def _build_sc_call(n, d, b, nblk_w):
    mesh = plsc.VectorSubcoreMesh(core_axis_name="c", subcore_axis_name="s")
    npc = d // 16
    ngr = _RP // 16

    @functools.partial(
        pl.kernel,
        out_type=(
            jax.ShapeDtypeStruct((_NC, b, d), jnp.float32),
            jax.ShapeDtypeStruct((_NC, b, 16), jnp.float32),
        ),
        mesh=mesh,
        compiler_params=pltpu.CompilerParams(
            use_tc_tiling_on_sc=False, needs_layout_passes=False),
        scratch_types=[
            pltpu.VMEM((_NBUF, _RP, d), jnp.float32),
            pltpu.VMEM((nblk_w, _RP), jnp.int32),
            pltpu.VMEM((b, d), jnp.float32),
            pltpu.VMEM((b, 16), jnp.float32),
            pltpu.VMEM((b,), jnp.int32),
            pltpu.SemaphoreType.DMA,
            pltpu.VMEM_SHARED((b, d), jnp.float32),
            pltpu.VMEM_SHARED((b, 16), jnp.float32),
        ],
    )
    def sc_call(feats_hbm, ids_hbm, psum_hbm, pcnt_hbm,
                rows_v, ids_v, acc_v, cnt_v, iota_v, sem,
                acc_sh, cnt_sh):
        c = lax.axis_index("c")
        s = lax.axis_index("s")
        wid = s * _NC + c

        zeros16 = jnp.zeros((16,), jnp.float32)
        ones16 = jnp.ones((16,), jnp.float32)
        iota16 = lax.iota(jnp.int32, 16)
        col_idx = [iota16 + cc * 16 for cc in range(npc)]
        grp_mask = [
            jnp.where(iota16 + g * 16 < _R, ones16, zeros16)
            for g in range(ngr)
        ]

        def fill_acc(r, carry):
            for cc in range(npc):
                acc_v[r, pl.ds(cc * 16, 16)] = zeros16
            cnt_v[r, :] = zeros16
            return carry

        lax.fori_loop(0, b, fill_acc, 0)

        def fill_ring_pad(r, carry):
            for bb in range(_NBUF):
                for cc in range(npc):
                    rows_v[bb, r, pl.ds(cc * 16, 16)] = zeros16
            return carry

        lax.fori_loop(_R, _RP, fill_ring_pad, 0)
        for r in range(b // 16):
            iota_v[pl.ds(r * 16, 16)] = iota16 + (r * 16)

        @pl.when(s == 0)
        def _():
            pltpu.sync_copy(acc_v, acc_sh)
            pltpu.sync_copy(cnt_v, cnt_sh)

        plsc.subcore_barrier()

        pltpu.sync_copy(ids_hbm.at[pl.ds(wid * nblk_w, nblk_w)], ids_v)

        def row_dma(kk, buf):
            row0 = (wid * nblk_w + kk) * _R
            return pltpu.async_copy(feats_hbm.at[pl.ds(row0, _R)],
                                    rows_v.at[buf, pl.ds(0, _R)], sem)

        for p in range(_NBUF - 1):
            row_dma(p, p)

        def flush(cur_seg, acc):
            segv = jnp.full((16,), cur_seg)
            for cc in range(npc):
                plsc.addupdate_scatter(acc_v, [segv, col_idx[cc]], acc[cc])

        def accumulate(acc, buf):
            for j in range(_RP):
                acc = tuple(
                    acc[cc] + rows_v[buf, j, pl.ds(cc * 16, 16)]
                    for cc in range(npc))
            return acc

        def process_block(kk, buf, state):
            pltpu.make_async_copy(
                feats_hbm.at[pl.ds(0, _R)],
                rows_v.at[buf, pl.ds(0, _R)], sem).wait()

            @pl.when(kk + _NBUF - 1 < nblk_w)
            def _():
                row_dma(kk + _NBUF - 1,
                        lax.rem(buf + _NBUF - 1, _NBUF))

            idsg = [ids_v[kk, pl.ds(g * 16, 16)] for g in range(ngr)]
            for g in range(ngr):
                plsc.addupdate_scatter(cnt_v, [idsg[g], iota16], grp_mask[g])

            mnv = idsg[0]
            mxv = idsg[0]
            for g in range(1, ngr):
                mnv = jnp.minimum(mnv, idsg[g])
                mxv = jnp.maximum(mxv, idsg[g])
            mnb = jnp.min(mnv)
            mxb = jnp.max(mxv)

            def uniform_blk(st):
                cur_seg0, acc0 = st

                def same_run(st2):
                    return cur_seg0, accumulate(st2[1], buf)

                def new_run(st2):
                    flush(cur_seg0, st2[1])
                    return mxb, accumulate((zeros16,) * npc, buf)

                return lax.cond(mxb == cur_seg0, same_run, new_run, st)

            def mixed_blk(st):
                flush(st[0], st[1])
                for g in range(ngr):
                    for j in range(16):
                        seg = jnp.take_along_axis(
                            idsg[g], jnp.full((16,), j, jnp.int32), axis=0)
                        for cc in range(npc):
                            vals = rows_v[buf, g * 16 + j,
                                          pl.ds(cc * 16, 16)]
                            plsc.addupdate_scatter(
                                acc_v, [seg, col_idx[cc]], vals)
                return mxb, (zeros16,) * npc

            return lax.cond(mnb == mxb, uniform_blk, mixed_blk, state)

        state = (jnp.min(ids_v[0, pl.ds(0, 16)]), (zeros16,) * npc)

        state = lax.fori_loop(
            0, nblk_w,
            lambda kk, st: process_block(kk, lax.rem(kk, _NBUF), st),
            state)

        flush(state[0], state[1])

        pltpu.sync_copy(acc_v, acc_sh.at[iota_v], add=True)
        pltpu.sync_copy(cnt_v, cnt_sh.at[iota_v], add=True)

        plsc.subcore_barrier()

        @pl.when(s == 0)
        def _():
            pltpu.sync_copy(acc_sh, psum_hbm.at[c])
            pltpu.sync_copy(cnt_sh, pcnt_hbm.at[c])

    return sc_call


def kernel(sfb, node_feats, segment_ids):
    n, d = node_feats.shape
    b = sfb.shape[0]
    nblk = n // _R
    nblk_w = nblk // (_NC * _NS)
    assert nblk * _R == n and nblk_w * _NC * _NS == nblk
    assert d % 16 == 0 and b % 16 == 0

    ids = segment_ids.astype(jnp.int32).reshape(nblk, _R)
    ids_pad = jnp.pad(ids, ((0, 0), (0, _RP - _R)), mode="edge")

    psum, pcnt = _build_sc_call(n, d, b, nblk_w)(node_feats, ids_pad)

    sums = psum[0] + psum[1]
    cnt = pcnt.sum(axis=(0, 2))
    g_feat = sums / jnp.maximum(cnt, 1.0)[:, None]
    return jnp.concatenate(
        (sfb, g_feat.reshape(b, d, 1, 1, 1)), axis=1)


# speedup vs baseline: 2.1841x; 2.1841x over previous
import functools

import jax
import jax.numpy as jnp
from jax import lax
from jax.experimental import pallas as pl
from jax.experimental.pallas import tpu as pltpu
from jax.experimental.pallas import tpu_sc as plsc

_NC = 2
_NS = 16
_R = 125
_RP = 128
_NBUF = 4


@functools.lru_cache(maxsize=None)
def _build_sc_call(n, d, b, nblk_w):
    mesh = plsc.VectorSubcoreMesh(core_axis_name="c", subcore_axis_name="s")
    npc = d // 16
    ngr = _RP // 16

    @functools.partial(
        pl.kernel,
        out_type=(
            jax.ShapeDtypeStruct((_NC, b, d), jnp.float32),
            jax.ShapeDtypeStruct((_NC, b, 16), jnp.float32),
        ),
        mesh=mesh,
        compiler_params=pltpu.CompilerParams(
            use_tc_tiling_on_sc=False, needs_layout_passes=False),
        scratch_types=[
            pltpu.VMEM((_NBUF, _RP, d), jnp.float32),
            pltpu.VMEM((nblk_w, _RP), jnp.int32),
            pltpu.VMEM((b, d), jnp.float32),
            pltpu.VMEM((b, 16), jnp.float32),
            pltpu.VMEM((b,), jnp.int32),
            pltpu.SemaphoreType.DMA,
            pltpu.SemaphoreType.DMA,
            pltpu.VMEM_SHARED((b, d), jnp.float32),
            pltpu.VMEM_SHARED((b, 16), jnp.float32),
        ],
    )
    def sc_call(feats_hbm, ids_hbm, psum_hbm, pcnt_hbm,
                rows_v, ids_v, zf_v, cnt_v, iota_v, sem, sem2,
                acc_sh, cnt_sh):
        c = lax.axis_index("c")
        s = lax.axis_index("s")
        wid = s * _NC + c

        zeros16 = jnp.zeros((16,), jnp.float32)
        ones16 = jnp.ones((16,), jnp.float32)
        iota16 = lax.iota(jnp.int32, 16)
        grp_mask = [
            jnp.where(iota16 + g * 16 < _R, ones16, zeros16)
            for g in range(ngr)
        ]

        def fill_zero(r, carry):
            for cc in range(npc):
                zf_v[r, pl.ds(cc * 16, 16)] = zeros16
            cnt_v[r, :] = zeros16
            return carry

        lax.fori_loop(0, b, fill_zero, 0)

        def fill_ring_pad(r, carry):
            for bb in range(_NBUF):
                for cc in range(npc):
                    rows_v[bb, r, pl.ds(cc * 16, 16)] = zeros16
            return carry

        lax.fori_loop(_R, _RP, fill_ring_pad, 0)
        for r in range(b // 16):
            iota_v[pl.ds(r * 16, 16)] = iota16 + (r * 16)

        @pl.when(s == 0)
        def _():
            pltpu.sync_copy(zf_v, acc_sh)
            pltpu.sync_copy(cnt_v.at[pl.ds(0, b)], cnt_sh)

        plsc.subcore_barrier()

        pltpu.sync_copy(ids_hbm.at[pl.ds(wid * nblk_w, nblk_w)], ids_v)

        def row_dma(kk, buf):
            row0 = (wid * nblk_w + kk) * _R
            return pltpu.async_copy(feats_hbm.at[pl.ds(row0, _R)],
                                    rows_v.at[buf, pl.ds(0, _R)], sem)

        def wait_one_scatter():
            pltpu.make_async_copy(
                rows_v.at[0], acc_sh.at[ids_v.at[0]], sem2).wait()

        for p in range(_NBUF - 1):
            row_dma(p, p)

        def process_block(kk, buf, carry):
            @pl.when(kk >= 1)
            def _():
                wait_one_scatter()

            @pl.when(kk + _NBUF - 1 < nblk_w)
            def _():
                row_dma(kk + _NBUF - 1, (buf + _NBUF - 1) % _NBUF)

            pltpu.make_async_copy(
                feats_hbm.at[pl.ds(0, _R)],
                rows_v.at[buf, pl.ds(0, _R)], sem).wait()

            pltpu.async_copy(rows_v.at[buf], acc_sh.at[ids_v.at[kk]],
                             sem2, add=True)

            for g in range(ngr):
                ids_vec = ids_v[kk, pl.ds(g * 16, 16)]
                plsc.addupdate_scatter(cnt_v, [ids_vec, iota16], grp_mask[g])
            return carry

        def ring_body(grp, carry):
            for b2 in range(_NBUF):
                kk = grp * _NBUF + b2

                @pl.when(kk < nblk_w)
                def _():
                    process_block(kk, b2, 0)
            return carry

        lax.fori_loop(0, (nblk_w + _NBUF - 1) // _NBUF, ring_body, 0)
        wait_one_scatter()

        pltpu.sync_copy(cnt_v, cnt_sh.at[iota_v], add=True)

        plsc.subcore_barrier()

        @pl.when(s == 0)
        def _():
            pltpu.sync_copy(acc_sh, psum_hbm.at[c])
            pltpu.sync_copy(cnt_sh, pcnt_hbm.at[c])

    return sc_call


def kernel(sfb, node_feats, segment_ids):
    n, d = node_feats.shape
    b = sfb.shape[0]
    nblk = n // _R
    nblk_w = nblk // (_NC * _NS)
    assert nblk * _R == n and nblk_w * _NC * _NS == nblk
    assert d % 16 == 0 and b % 16 == 0

    ids = segment_ids.astype(jnp.int32).reshape(nblk, _R)
    ids_pad = jnp.pad(ids, ((0, 0), (0, _RP - _R)), mode="edge")

    psum, pcnt = _build_sc_call(n, d, b, nblk_w)(node_feats, ids_pad)

    sums = psum[0] + psum[1]
    cnt = pcnt.sum(axis=(0, 2))
    g_feat = sums / jnp.maximum(cnt, 1.0)[:, None]
    return jnp.concatenate(
        (sfb, g_feat.reshape(b, d, 1, 1, 1)), axis=1)


# speedup vs baseline: 3.4128x; 1.5625x over previous
import functools

import jax
import jax.numpy as jnp
from jax import lax
from jax.experimental import pallas as pl
from jax.experimental.pallas import tpu as pltpu
from jax.experimental.pallas import tpu_sc as plsc

_NC = 2
_NS = 16
_R = 125
_RP = 128
_NBUF = 4


@functools.lru_cache(maxsize=None)
def _build_sc_call(n, d, b, nblk_w):
    mesh = plsc.VectorSubcoreMesh(core_axis_name="c", subcore_axis_name="s")
    npc = d // 16

    @functools.partial(
        pl.kernel,
        out_type=(
            jax.ShapeDtypeStruct((_NC, b, d), jnp.float32),
            jax.ShapeDtypeStruct((_NC, b, 16), jnp.float32),
        ),
        mesh=mesh,
        compiler_params=pltpu.CompilerParams(
            use_tc_tiling_on_sc=False, needs_layout_passes=False),
        scratch_types=[
            pltpu.VMEM((_NBUF, _RP, d), jnp.float32),
            pltpu.VMEM((nblk_w, _RP), jnp.int32),
            pltpu.VMEM((b, d), jnp.float32),
            pltpu.VMEM((b, 16), jnp.float32),
            pltpu.VMEM((b,), jnp.int32),
            pltpu.SemaphoreType.DMA,
            pltpu.VMEM_SHARED((b, d), jnp.float32),
            pltpu.VMEM_SHARED((b, 16), jnp.float32),
        ],
    )
    def sc_call(feats_hbm, ids_hbm, psum_hbm, pcnt_hbm,
                rows_v, ids_v, acc_v, cnt_v, iota_v, sem,
                acc_sh, cnt_sh):
        c = lax.axis_index("c")
        s = lax.axis_index("s")
        wid = s * _NC + c

        zeros16 = jnp.zeros((16,), jnp.float32)
        ones16 = jnp.ones((16,), jnp.float32)
        iota16 = lax.iota(jnp.int32, 16)
        col_idx = [iota16 + cc * 16 for cc in range(npc)]

        def fill_acc(r, carry):
            for cc in range(npc):
                acc_v[r, pl.ds(cc * 16, 16)] = zeros16
            cnt_v[r, :] = zeros16
            return carry

        lax.fori_loop(0, b, fill_acc, 0)

        def fill_ring_pad(r, carry):
            for bb in range(_NBUF):
                for cc in range(npc):
                    rows_v[bb, r, pl.ds(cc * 16, 16)] = zeros16
            return carry

        lax.fori_loop(_R, _RP, fill_ring_pad, 0)
        for r in range(b // 16):
            iota_v[pl.ds(r * 16, 16)] = iota16 + (r * 16)

        @pl.when(s == 0)
        def _():
            pltpu.sync_copy(acc_v, acc_sh)
            pltpu.sync_copy(cnt_v, cnt_sh)

        plsc.subcore_barrier()

        pltpu.sync_copy(ids_hbm.at[pl.ds(wid * nblk_w, nblk_w)], ids_v)

        def row_dma(kk, buf):
            row0 = (wid * nblk_w + kk) * _R
            return pltpu.async_copy(feats_hbm.at[pl.ds(row0, _R)],
                                    rows_v.at[buf, pl.ds(0, _R)], sem)

        for p in range(_NBUF - 1):
            row_dma(p, p)

        def flush(cur_seg, acc):
            segv = jnp.full((16,), cur_seg)
            for cc in range(npc):
                plsc.addupdate_scatter(acc_v, [segv, col_idx[cc]], acc[cc])

        def accumulate(acc, buf, g):
            for j in range(16):
                acc = tuple(
                    acc[cc] + rows_v[buf, g * 16 + j, pl.ds(cc * 16, 16)]
                    for cc in range(npc))
            return acc

        def group_step(kk, buf, g, state):
            cur_seg, acc = state
            ids_vec = ids_v[kk, pl.ds(g * 16, 16)]
            valid = (iota16 + g * 16) < _R
            plsc.addupdate_scatter(
                cnt_v, [ids_vec, iota16],
                jnp.where(valid, ones16, zeros16))

            mn = jnp.min(ids_vec)
            mx = jnp.max(ids_vec)

            def uniform_case(st):
                cur_seg0, acc0 = st

                def same_run(st2):
                    return cur_seg0, accumulate(st2[1], buf, g)

                def new_run(st2):
                    flush(cur_seg0, st2[1])
                    return mx, accumulate((zeros16,) * npc, buf, g)

                return lax.cond(mx == cur_seg0, same_run, new_run, st)

            def mixed_case(st):
                flush(st[0], st[1])
                for j in range(16):
                    seg = jnp.take_along_axis(
                        ids_vec, jnp.full((16,), j, jnp.int32), axis=0)
                    for cc in range(npc):
                        vals = rows_v[buf, g * 16 + j, pl.ds(cc * 16, 16)]
                        plsc.addupdate_scatter(acc_v, [seg, col_idx[cc]], vals)
                return mx, (zeros16,) * npc

            return lax.cond(mn == mx, uniform_case, mixed_case,
                            (cur_seg, acc))

        def process_block(kk, buf, state):
            pltpu.make_async_copy(
                feats_hbm.at[pl.ds(0, _R)],
                rows_v.at[buf, pl.ds(0, _R)], sem).wait()

            @pl.when(kk + _NBUF - 1 < nblk_w)
            def _():
                row_dma(kk + _NBUF - 1, (buf + _NBUF - 1) % _NBUF)

            return lax.fori_loop(
                0, _RP // 16,
                lambda g, st: group_step(kk, buf, g, st), state)

        state = (jnp.min(ids_v[0, pl.ds(0, 16)]), (zeros16,) * npc)

        def ring_body(grp, st):
            for b2 in range(_NBUF):
                st = process_block(grp * _NBUF + b2, b2, st)
            return st

        nfull = nblk_w // _NBUF
        state = lax.fori_loop(0, nfull, ring_body, state)
        for kk in range(nfull * _NBUF, nblk_w):
            state = process_block(kk, kk % _NBUF, state)

        flush(state[0], state[1])

        pltpu.sync_copy(acc_v, acc_sh.at[iota_v], add=True)
        pltpu.sync_copy(cnt_v, cnt_sh.at[iota_v], add=True)

        plsc.subcore_barrier()

        @pl.when(s == 0)
        def _():
            pltpu.sync_copy(acc_sh, psum_hbm.at[c])
            pltpu.sync_copy(cnt_sh, pcnt_hbm.at[c])

    return sc_call


_SC_BLOCKS_W = 12
_TC_BLK = 2000


@functools.lru_cache(maxsize=None)
def _build_tc_call(d, b, blk0, nblk_tc):
    blk = _TC_BLK

    def tc_kernel(ids_ref, feats_ref, sum_ref, cnt_ref):
        i = pl.program_id(0)
        ids_blk = ids_ref[0, 0, :]
        onehot = (lax.broadcasted_iota(jnp.int32, (b, blk), 0)
                  == ids_blk[None, :]).astype(jnp.float32)
        psum = jnp.dot(onehot, feats_ref[...],
                       preferred_element_type=jnp.float32)
        pcnt = jnp.broadcast_to(
            jnp.sum(onehot, axis=1, keepdims=True), (b, d))

        @pl.when(i == 0)
        def _():
            sum_ref[...] = psum
            cnt_ref[...] = pcnt

        @pl.when(i != 0)
        def _():
            sum_ref[...] += psum
            cnt_ref[...] += pcnt

    return pl.pallas_call(
        tc_kernel,
        grid=(nblk_tc,),
        in_specs=[
            pl.BlockSpec((1, 1, blk), lambda i: (i, 0, 0)),
            pl.BlockSpec((blk, d), lambda i: (blk0 + i, 0)),
        ],
        out_specs=[
            pl.BlockSpec((b, d), lambda i: (0, 0)),
            pl.BlockSpec((b, d), lambda i: (0, 0)),
        ],
        out_shape=[
            jax.ShapeDtypeStruct((b, d), jnp.float32),
            jax.ShapeDtypeStruct((b, d), jnp.float32),
        ],
        compiler_params=pltpu.CompilerParams(
            dimension_semantics=("arbitrary",)),
    )


def kernel(sfb, node_feats, segment_ids):
    n, d = node_feats.shape
    b = sfb.shape[0]
    n_sc = _NC * _NS * _SC_BLOCKS_W * _R
    n_tc = n - n_sc
    assert n_tc % _TC_BLK == 0 and n_sc % _TC_BLK == 0
    assert d % 16 == 0 and b % 16 == 0

    ids32 = segment_ids.astype(jnp.int32)
    ids_sc = ids32[:n_sc].reshape(n_sc // _R, _R)
    ids_pad = jnp.pad(ids_sc, ((0, 0), (0, _RP - _R)), mode="edge")
    ids_tc = ids32[n_sc:].reshape(n_tc // _TC_BLK, 1, _TC_BLK)

    psum, pcnt = _build_sc_call(n, d, b, _SC_BLOCKS_W)(node_feats, ids_pad)
    tsum, tcnt = _build_tc_call(d, b, n_sc // _TC_BLK,
                                n_tc // _TC_BLK)(ids_tc, node_feats)

    sums = psum[0] + psum[1] + tsum
    cnt = pcnt.sum(axis=(0, 2)) + tcnt[:, 0]
    g_feat = sums / jnp.maximum(cnt, 1.0)[:, None]
    return jnp.concatenate(
        (sfb, g_feat.reshape(b, d, 1, 1, 1)), axis=1)
